# initial kernel scaffold (unmeasured)
import jax
import jax.numpy as jnp
from jax import lax
from jax.experimental import pallas as pl
from jax.experimental.pallas import tpu as pltpu

N_X = 2
N_Y = 4
N_Z = 4
N_REPL = N_Y * N_Z
N_ROUNDS = 5


def kernel(x, dy, gamma):
    del gamma
    m, d = x.shape
    rows = m // N_REPL

    def body(x_hbm, dy_hbm, out_ref, xb, dyb, acc, recv,
             in_sems, send_sems, recv_sems):
        my_x = lax.axis_index("x")
        my_y = lax.axis_index("y")
        my_z = lax.axis_index("z")

        partners = [
            (1 - my_x, my_y, my_z),
            (my_x, my_y ^ 1, my_z),
            (my_x, my_y ^ 2, my_z),
            (my_x, my_y, my_z ^ 1),
            (my_x, my_y, my_z ^ 2),
        ]

        barrier = pltpu.get_barrier_semaphore()
        for p in partners:
            pl.semaphore_signal(
                barrier, inc=1, device_id=p,
                device_id_type=pl.DeviceIdType.MESH,
            )

        r = my_y * N_Z + my_z
        cp_x = pltpu.make_async_copy(
            x_hbm.at[pl.ds(r * rows, rows), :], xb, in_sems.at[0])
        cp_dy = pltpu.make_async_copy(
            dy_hbm.at[pl.ds(r * rows, rows), :], dyb, in_sems.at[1])
        cp_x.start()
        cp_dy.start()
        cp_x.wait()
        cp_dy.wait()

        xv = xb[...]
        dyv = dyb[...]
        mu = jnp.mean(xv, axis=1, keepdims=True)
        var = jnp.mean(xv * xv, axis=1, keepdims=True) - mu * mu
        rstd = lax.rsqrt(var + 1e-5)
        xhat = (xv - mu) * rstd
        dg = jnp.sum(dyv * xhat, axis=0, keepdims=True)
        db = jnp.sum(dyv, axis=0, keepdims=True)
        acc[...] = jnp.concatenate([dg, db], axis=0)

        pl.semaphore_wait(barrier, N_ROUNDS)

        for rnd, p in enumerate(partners):
            rdma = pltpu.make_async_remote_copy(
                src_ref=acc,
                dst_ref=recv.at[rnd],
                send_sem=send_sems.at[rnd],
                recv_sem=recv_sems.at[rnd],
                device_id=p,
                device_id_type=pl.DeviceIdType.MESH,
            )
            rdma.start()
            rdma.wait()
            acc[...] = acc[...] + recv[rnd]

        out_ref[...] = acc[...]

    return pl.pallas_call(
        body,
        out_shape=jax.ShapeDtypeStruct((2, d), jnp.float32),
        in_specs=[
            pl.BlockSpec(memory_space=pltpu.ANY),
            pl.BlockSpec(memory_space=pltpu.ANY),
        ],
        out_specs=pl.BlockSpec(memory_space=pltpu.VMEM),
        scratch_shapes=[
            pltpu.VMEM((rows, d), jnp.float32),
            pltpu.VMEM((rows, d), jnp.float32),
            pltpu.VMEM((2, d), jnp.float32),
            pltpu.VMEM((N_ROUNDS, 2, d), jnp.float32),
            pltpu.SemaphoreType.DMA((2,)),
            pltpu.SemaphoreType.DMA((N_ROUNDS,)),
            pltpu.SemaphoreType.DMA((N_ROUNDS,)),
        ],
        compiler_params=pltpu.CompilerParams(collective_id=0),
    )(x, dy)


# baseline (device time: 19070 ns/iter reference)
import jax
import jax.numpy as jnp
from jax import lax
from jax.experimental import pallas as pl
from jax.experimental.pallas import tpu as pltpu

N_X = 2
N_Y = 4
N_Z = 4
N_REPL = N_Y * N_Z
N_ROUNDS = 5


def kernel(x, dy, gamma):
    del gamma
    m, d = x.shape
    rows = m // N_REPL

    def body(x_hbm, dy_hbm, out_ref, xb, dyb, acc, recv,
             in_sems, send_sems, recv_sems):
        my_x = lax.axis_index("x")
        my_y = lax.axis_index("y")
        my_z = lax.axis_index("z")

        partners = [
            (1 - my_x, my_y, my_z),
            (my_x, my_y ^ 1, my_z),
            (my_x, my_y ^ 2, my_z),
            (my_x, my_y, my_z ^ 1),
            (my_x, my_y, my_z ^ 2),
        ]

        barrier = pltpu.get_barrier_semaphore()
        for p in partners:
            pl.semaphore_signal(
                barrier, inc=1, device_id=p,
                device_id_type=pl.DeviceIdType.MESH,
            )

        r = my_y * N_Z + my_z
        cp_x = pltpu.make_async_copy(
            x_hbm.at[pl.ds(r * rows, rows), :], xb, in_sems.at[0])
        cp_dy = pltpu.make_async_copy(
            dy_hbm.at[pl.ds(r * rows, rows), :], dyb, in_sems.at[1])
        cp_x.start()
        cp_dy.start()
        cp_x.wait()
        cp_dy.wait()

        xv = xb[...]
        dyv = dyb[...]
        mu = jnp.mean(xv, axis=1, keepdims=True)
        var = jnp.mean(xv * xv, axis=1, keepdims=True) - mu * mu
        rstd = lax.rsqrt(var + 1e-5)
        xhat = (xv - mu) * rstd
        dg = jnp.sum(dyv * xhat, axis=0, keepdims=True)
        db = jnp.sum(dyv, axis=0, keepdims=True)
        acc[...] = jnp.concatenate([dg, db], axis=0)

        pl.semaphore_wait(barrier, N_ROUNDS)

        for rnd, p in enumerate(partners):
            rdma = pltpu.make_async_remote_copy(
                src_ref=acc,
                dst_ref=recv.at[rnd],
                send_sem=send_sems.at[rnd],
                recv_sem=recv_sems.at[rnd],
                device_id=p,
                device_id_type=pl.DeviceIdType.MESH,
            )
            rdma.start()
            rdma.wait()
            acc[...] = acc[...] + recv[rnd]

        out_ref[...] = acc[...]

    return pl.pallas_call(
        body,
        out_shape=jax.ShapeDtypeStruct((2, d), jnp.float32),
        in_specs=[
            pl.BlockSpec(memory_space=pl.ANY),
            pl.BlockSpec(memory_space=pl.ANY),
        ],
        out_specs=pl.BlockSpec(memory_space=pltpu.VMEM),
        scratch_shapes=[
            pltpu.VMEM((rows, d), jnp.float32),
            pltpu.VMEM((rows, d), jnp.float32),
            pltpu.VMEM((2, d), jnp.float32),
            pltpu.VMEM((N_ROUNDS, 2, d), jnp.float32),
            pltpu.SemaphoreType.DMA((2,)),
            pltpu.SemaphoreType.DMA((N_ROUNDS,)),
            pltpu.SemaphoreType.DMA((N_ROUNDS,)),
        ],
        compiler_params=pltpu.CompilerParams(collective_id=0),
    )(x, dy)


# device time: 17181 ns/iter; 1.1099x vs baseline; 1.1099x over previous
import jax
import jax.numpy as jnp
from jax import lax
from jax.experimental import pallas as pl
from jax.experimental.pallas import tpu as pltpu

N_X = 2
N_Y = 4
N_Z = 4
N_DEV = N_X * N_Y * N_Z
N_REPL = N_Y * N_Z

OFFSETS = [
    (dx, dy_, dz)
    for dx in range(N_X)
    for dy_ in range(N_Y)
    for dz in range(N_Z)
    if (dx, dy_, dz) != (0, 0, 0)
]


def kernel(x, dy, gamma):
    del gamma
    m, d = x.shape
    rows = m // N_REPL

    def body(x_hbm, dy_hbm, out_ref, xb, dyb, acc, recv,
             in_sems, send_sems, recv_sems):
        my_x = lax.axis_index("x")
        my_y = lax.axis_index("y")
        my_z = lax.axis_index("z")
        my_lid = (my_x * N_Y + my_y) * N_Z + my_z

        peers = [
            ((my_x + dx) % N_X, (my_y + dy_) % N_Y, (my_z + dz) % N_Z)
            for dx, dy_, dz in OFFSETS
        ]

        barrier = pltpu.get_barrier_semaphore()
        for p in peers:
            pl.semaphore_signal(
                barrier, inc=1, device_id=p,
                device_id_type=pl.DeviceIdType.MESH,
            )

        r = my_y * N_Z + my_z
        cp_x = pltpu.make_async_copy(
            x_hbm.at[pl.ds(r * rows, rows), :], xb, in_sems.at[0])
        cp_dy = pltpu.make_async_copy(
            dy_hbm.at[pl.ds(r * rows, rows), :], dyb, in_sems.at[1])
        cp_x.start()
        cp_dy.start()
        cp_x.wait()
        cp_dy.wait()

        xv = xb[...]
        dyv = dyb[...]
        mu = jnp.mean(xv, axis=1, keepdims=True)
        var = jnp.mean(xv * xv, axis=1, keepdims=True) - mu * mu
        rstd = lax.rsqrt(var + 1e-5)
        xhat = (xv - mu) * rstd
        dg = jnp.sum(dyv * xhat, axis=0, keepdims=True)
        db = jnp.sum(dyv, axis=0, keepdims=True)
        partial = jnp.concatenate([dg, db], axis=0)
        acc[...] = partial
        recv[my_lid, :, :] = partial

        pl.semaphore_wait(barrier, len(peers))

        rdmas = []
        for i, p in enumerate(peers):
            rdma = pltpu.make_async_remote_copy(
                src_ref=acc,
                dst_ref=recv.at[my_lid],
                send_sem=send_sems.at[i],
                recv_sem=recv_sems.at[my_lid],
                device_id=p,
                device_id_type=pl.DeviceIdType.MESH,
            )
            rdma.start()
            rdmas.append(rdma)

        for s in range(N_DEV):
            @pl.when(s != my_lid)
            def _():
                pltpu.make_async_remote_copy(
                    src_ref=acc,
                    dst_ref=recv.at[s],
                    send_sem=send_sems.at[0],
                    recv_sem=recv_sems.at[s],
                    device_id=(my_x, my_y, my_z),
                    device_id_type=pl.DeviceIdType.MESH,
                ).wait_recv()

        out_ref[...] = jnp.sum(recv[...], axis=0)

        for rdma in rdmas:
            rdma.wait_send()

    return pl.pallas_call(
        body,
        out_shape=jax.ShapeDtypeStruct((2, d), jnp.float32),
        in_specs=[
            pl.BlockSpec(memory_space=pl.ANY),
            pl.BlockSpec(memory_space=pl.ANY),
        ],
        out_specs=pl.BlockSpec(memory_space=pltpu.VMEM),
        scratch_shapes=[
            pltpu.VMEM((rows, d), jnp.float32),
            pltpu.VMEM((rows, d), jnp.float32),
            pltpu.VMEM((2, d), jnp.float32),
            pltpu.VMEM((N_DEV, 2, d), jnp.float32),
            pltpu.SemaphoreType.DMA((2,)),
            pltpu.SemaphoreType.DMA((len(OFFSETS),)),
            pltpu.SemaphoreType.DMA((N_DEV,)),
        ],
        compiler_params=pltpu.CompilerParams(collective_id=0),
    )(x, dy)


# device time: 15520 ns/iter; 1.2287x vs baseline; 1.1070x over previous
import jax
import jax.numpy as jnp
from jax import lax
from jax.experimental import pallas as pl
from jax.experimental.pallas import tpu as pltpu

N_X = 2
N_Y = 4
N_Z = 4
N_XY = N_X * N_Y
N_REPL = N_Y * N_Z

OFFSETS_XY = [
    (dx, dy_) for dx in range(N_X) for dy_ in range(N_Y) if (dx, dy_) != (0, 0)
]
OFFSETS_Z = [dz for dz in range(1, N_Z)]


def kernel(x, dy, gamma):
    del gamma
    m, d = x.shape
    rows = m // N_REPL

    def body(x_hbm, dy_hbm, out_ref, xb, dyb, acc, recv_a, recv_b,
             in_sems, send_sems_a, recv_sems_a, send_sems_b, recv_sems_b):
        my_x = lax.axis_index("x")
        my_y = lax.axis_index("y")
        my_z = lax.axis_index("z")
        gid = my_x * N_Y + my_y

        peers_a = [
            ((my_x + dx) % N_X, (my_y + dy_) % N_Y, my_z)
            for dx, dy_ in OFFSETS_XY
        ]
        peers_b = [(my_x, my_y, (my_z + dz) % N_Z) for dz in OFFSETS_Z]

        barrier = pltpu.get_barrier_semaphore()
        for p in peers_a + peers_b:
            pl.semaphore_signal(
                barrier, inc=1, device_id=p,
                device_id_type=pl.DeviceIdType.MESH,
            )

        r = my_y * N_Z + my_z
        cp_x = pltpu.make_async_copy(
            x_hbm.at[pl.ds(r * rows, rows), :], xb, in_sems.at[0])
        cp_dy = pltpu.make_async_copy(
            dy_hbm.at[pl.ds(r * rows, rows), :], dyb, in_sems.at[1])
        cp_x.start()
        cp_dy.start()
        cp_x.wait()
        cp_dy.wait()

        xv = xb[...]
        dyv = dyb[...]
        mu = jnp.mean(xv, axis=1, keepdims=True)
        var = jnp.mean(xv * xv, axis=1, keepdims=True) - mu * mu
        rstd = lax.rsqrt(var + 1e-5)
        xhat = (xv - mu) * rstd
        dg = jnp.sum(dyv * xhat, axis=0, keepdims=True)
        db = jnp.sum(dyv, axis=0, keepdims=True)
        partial = jnp.concatenate([dg, db], axis=0)
        acc[...] = partial
        recv_a[gid, :, :] = partial

        pl.semaphore_wait(barrier, len(peers_a) + len(peers_b))

        rdmas_a = []
        for i, p in enumerate(peers_a):
            rdma = pltpu.make_async_remote_copy(
                src_ref=acc,
                dst_ref=recv_a.at[gid],
                send_sem=send_sems_a.at[i],
                recv_sem=recv_sems_a.at[gid],
                device_id=p,
                device_id_type=pl.DeviceIdType.MESH,
            )
            rdma.start()
            rdmas_a.append(rdma)

        for s in range(N_XY):
            @pl.when(s != gid)
            def _():
                pltpu.make_async_remote_copy(
                    src_ref=acc,
                    dst_ref=recv_a.at[s],
                    send_sem=send_sems_a.at[0],
                    recv_sem=recv_sems_a.at[s],
                    device_id=(my_x, my_y, my_z),
                    device_id_type=pl.DeviceIdType.MESH,
                ).wait_recv()

        acc[...] = jnp.sum(recv_a[...], axis=0)
        recv_b[my_z, :, :] = acc[...]

        rdmas_b = []
        for i, p in enumerate(peers_b):
            rdma = pltpu.make_async_remote_copy(
                src_ref=acc,
                dst_ref=recv_b.at[my_z],
                send_sem=send_sems_b.at[i],
                recv_sem=recv_sems_b.at[my_z],
                device_id=p,
                device_id_type=pl.DeviceIdType.MESH,
            )
            rdma.start()
            rdmas_b.append(rdma)

        for s in range(N_Z):
            @pl.when(s != my_z)
            def _():
                pltpu.make_async_remote_copy(
                    src_ref=acc,
                    dst_ref=recv_b.at[s],
                    send_sem=send_sems_b.at[0],
                    recv_sem=recv_sems_b.at[s],
                    device_id=(my_x, my_y, my_z),
                    device_id_type=pl.DeviceIdType.MESH,
                ).wait_recv()

        out_ref[...] = jnp.sum(recv_b[...], axis=0)

        for rdma in rdmas_a + rdmas_b:
            rdma.wait_send()

    return pl.pallas_call(
        body,
        out_shape=jax.ShapeDtypeStruct((2, d), jnp.float32),
        in_specs=[
            pl.BlockSpec(memory_space=pl.ANY),
            pl.BlockSpec(memory_space=pl.ANY),
        ],
        out_specs=pl.BlockSpec(memory_space=pltpu.VMEM),
        scratch_shapes=[
            pltpu.VMEM((rows, d), jnp.float32),
            pltpu.VMEM((rows, d), jnp.float32),
            pltpu.VMEM((2, d), jnp.float32),
            pltpu.VMEM((N_XY, 2, d), jnp.float32),
            pltpu.VMEM((N_Z, 2, d), jnp.float32),
            pltpu.SemaphoreType.DMA((2,)),
            pltpu.SemaphoreType.DMA((len(OFFSETS_XY),)),
            pltpu.SemaphoreType.DMA((N_XY,)),
            pltpu.SemaphoreType.DMA((len(OFFSETS_Z),)),
            pltpu.SemaphoreType.DMA((N_Z,)),
        ],
        compiler_params=pltpu.CompilerParams(collective_id=0),
    )(x, dy)


# device time: 15123 ns/iter; 1.2610x vs baseline; 1.0263x over previous
import jax
import jax.numpy as jnp
from jax import lax
from jax.experimental import pallas as pl
from jax.experimental.pallas import tpu as pltpu

N_X = 2
N_Y = 4
N_Z = 4
N_XY = N_X * N_Y
N_REPL = N_Y * N_Z

OFFSETS_XY = [
    (dx, dy_) for dx in range(N_X) for dy_ in range(N_Y) if (dx, dy_) != (0, 0)
]
OFFSETS_Z = [dz for dz in range(1, N_Z)]


def kernel(x, dy, gamma):
    del gamma
    m, d = x.shape
    rows = m // N_REPL

    def body(x_hbm, dy_hbm, out_ref, xb, dyb, acc, recv_a, recv_b,
             in_sems, send_sems_a, recv_sems_a, send_sems_b, recv_sems_b):
        my_x = lax.axis_index("x")
        my_y = lax.axis_index("y")
        my_z = lax.axis_index("z")
        gid = my_x * N_Y + my_y

        peers_a = [
            ((my_x + dx) % N_X, (my_y + dy_) % N_Y, my_z)
            for dx, dy_ in OFFSETS_XY
        ]
        peers_b = [(my_x, my_y, (my_z + dz) % N_Z) for dz in OFFSETS_Z]

        barrier = pltpu.get_barrier_semaphore()
        for p in peers_a + peers_b:
            pl.semaphore_signal(
                barrier, inc=1, device_id=p,
                device_id_type=pl.DeviceIdType.MESH,
            )

        r = my_y * N_Z + my_z
        cp_x = pltpu.make_async_copy(
            x_hbm.at[pl.ds(r * rows, rows), :], xb, in_sems.at[0])
        cp_dy = pltpu.make_async_copy(
            dy_hbm.at[pl.ds(r * rows, rows), :], dyb, in_sems.at[1])
        cp_x.start()
        cp_dy.start()
        cp_x.wait()
        cp_dy.wait()

        xv = xb[...]
        dyv = dyb[...]
        mu = jnp.mean(xv, axis=1, keepdims=True)
        var = jnp.mean(xv * xv, axis=1, keepdims=True) - mu * mu
        rstd = lax.rsqrt(var + 1e-5)
        xhat = (xv - mu) * rstd
        dg = jnp.sum(dyv * xhat, axis=0, keepdims=True)
        db = jnp.sum(dyv, axis=0, keepdims=True)
        partial = jnp.concatenate([dg, db], axis=0).astype(jnp.bfloat16)
        acc[...] = partial
        recv_a[gid, :, :] = partial

        pl.semaphore_wait(barrier, len(peers_a) + len(peers_b))

        rdmas_a = []
        for i, p in enumerate(peers_a):
            rdma = pltpu.make_async_remote_copy(
                src_ref=acc,
                dst_ref=recv_a.at[gid],
                send_sem=send_sems_a.at[i],
                recv_sem=recv_sems_a.at[gid],
                device_id=p,
                device_id_type=pl.DeviceIdType.MESH,
            )
            rdma.start()
            rdmas_a.append(rdma)

        for s in range(N_XY):
            @pl.when(s != gid)
            def _():
                pltpu.make_async_remote_copy(
                    src_ref=acc,
                    dst_ref=recv_a.at[s],
                    send_sem=send_sems_a.at[0],
                    recv_sem=recv_sems_a.at[s],
                    device_id=(my_x, my_y, my_z),
                    device_id_type=pl.DeviceIdType.MESH,
                ).wait_recv()

        plane = jnp.sum(recv_a[...].astype(jnp.float32), axis=0)
        acc[...] = plane.astype(jnp.bfloat16)
        recv_b[my_z, :, :] = acc[...]

        rdmas_b = []
        for i, p in enumerate(peers_b):
            rdma = pltpu.make_async_remote_copy(
                src_ref=acc,
                dst_ref=recv_b.at[my_z],
                send_sem=send_sems_b.at[i],
                recv_sem=recv_sems_b.at[my_z],
                device_id=p,
                device_id_type=pl.DeviceIdType.MESH,
            )
            rdma.start()
            rdmas_b.append(rdma)

        for s in range(N_Z):
            @pl.when(s != my_z)
            def _():
                pltpu.make_async_remote_copy(
                    src_ref=acc,
                    dst_ref=recv_b.at[s],
                    send_sem=send_sems_b.at[0],
                    recv_sem=recv_sems_b.at[s],
                    device_id=(my_x, my_y, my_z),
                    device_id_type=pl.DeviceIdType.MESH,
                ).wait_recv()

        out_ref[...] = jnp.sum(recv_b[...].astype(jnp.float32), axis=0)

        for rdma in rdmas_a + rdmas_b:
            rdma.wait_send()

    return pl.pallas_call(
        body,
        out_shape=jax.ShapeDtypeStruct((2, d), jnp.float32),
        in_specs=[
            pl.BlockSpec(memory_space=pl.ANY),
            pl.BlockSpec(memory_space=pl.ANY),
        ],
        out_specs=pl.BlockSpec(memory_space=pltpu.VMEM),
        scratch_shapes=[
            pltpu.VMEM((rows, d), jnp.float32),
            pltpu.VMEM((rows, d), jnp.float32),
            pltpu.VMEM((2, d), jnp.bfloat16),
            pltpu.VMEM((N_XY, 2, d), jnp.bfloat16),
            pltpu.VMEM((N_Z, 2, d), jnp.bfloat16),
            pltpu.SemaphoreType.DMA((2,)),
            pltpu.SemaphoreType.DMA((len(OFFSETS_XY),)),
            pltpu.SemaphoreType.DMA((N_XY,)),
            pltpu.SemaphoreType.DMA((len(OFFSETS_Z),)),
            pltpu.SemaphoreType.DMA((N_Z,)),
        ],
        compiler_params=pltpu.CompilerParams(collective_id=0),
    )(x, dy)


# device time: 14737 ns/iter; 1.2940x vs baseline; 1.0262x over previous
import jax
import jax.numpy as jnp
from jax import lax
from jax.experimental import pallas as pl
from jax.experimental.pallas import tpu as pltpu

N_X = 2
N_Y = 4
N_Z = 4
N_XY = N_X * N_Y
N_REPL = N_Y * N_Z

OFFSETS_XY = [
    (dx, dy_) for dx in range(N_X) for dy_ in range(N_Y) if (dx, dy_) != (0, 0)
]
OFFSETS_Z = [dz for dz in range(1, N_Z)]


def kernel(x, dy, gamma):
    del gamma
    m, d = x.shape
    rows = m // N_REPL

    def body(x_hbm, dy_hbm, out_ref, xb, dyb, acc, recv_a, recv_b,
             in_sems, send_sems_a, recv_sems_a, send_sems_b, recv_sems_b):
        my_x = lax.axis_index("x")
        my_y = lax.axis_index("y")
        my_z = lax.axis_index("z")
        gid = my_x * N_Y + my_y

        peers_a = [
            ((my_x + dx) % N_X, (my_y + dy_) % N_Y, my_z)
            for dx, dy_ in OFFSETS_XY
        ]
        peers_b = [(my_x, my_y, (my_z + dz) % N_Z) for dz in OFFSETS_Z]

        barrier = pltpu.get_barrier_semaphore()
        for p in peers_a + peers_b:
            pl.semaphore_signal(
                barrier, inc=1, device_id=p,
                device_id_type=pl.DeviceIdType.MESH,
            )

        r = my_y * N_Z + my_z
        half = rows // 2
        cps = []
        for c in range(2):
            cp_x = pltpu.make_async_copy(
                x_hbm.at[pl.ds(r * rows + c * half, half), :],
                xb.at[pl.ds(c * half, half), :], in_sems.at[2 * c])
            cp_dy = pltpu.make_async_copy(
                dy_hbm.at[pl.ds(r * rows + c * half, half), :],
                dyb.at[pl.ds(c * half, half), :], in_sems.at[2 * c + 1])
            cp_x.start()
            cp_dy.start()
            cps.append((cp_x, cp_dy))

        partial = None
        for c in range(2):
            cps[c][0].wait()
            cps[c][1].wait()
            xv = xb[pl.ds(c * half, half), :]
            dyv = dyb[pl.ds(c * half, half), :]
            mu = jnp.mean(xv, axis=1, keepdims=True)
            var = jnp.mean(xv * xv, axis=1, keepdims=True) - mu * mu
            rstd = lax.rsqrt(var + 1e-5)
            xhat = (xv - mu) * rstd
            dg = jnp.sum(dyv * xhat, axis=0, keepdims=True)
            db = jnp.sum(dyv, axis=0, keepdims=True)
            chunk = jnp.concatenate([dg, db], axis=0)
            partial = chunk if partial is None else partial + chunk
        partial = partial.astype(jnp.bfloat16)
        acc[...] = partial

        pl.semaphore_wait(barrier, len(peers_a) + len(peers_b))

        rdmas_a = []
        for i, p in enumerate(peers_a):
            rdma = pltpu.make_async_remote_copy(
                src_ref=acc,
                dst_ref=recv_a.at[gid],
                send_sem=send_sems_a.at[i],
                recv_sem=recv_sems_a.at[gid],
                device_id=p,
                device_id_type=pl.DeviceIdType.MESH,
            )
            rdma.start()
            rdmas_a.append(rdma)

        recv_a[gid, :, :] = partial

        for s in range(N_XY):
            @pl.when(s != gid)
            def _():
                pltpu.make_async_remote_copy(
                    src_ref=acc,
                    dst_ref=recv_a.at[s],
                    send_sem=send_sems_a.at[0],
                    recv_sem=recv_sems_a.at[s],
                    device_id=(my_x, my_y, my_z),
                    device_id_type=pl.DeviceIdType.MESH,
                ).wait_recv()

        plane = jnp.sum(recv_a[...].astype(jnp.float32), axis=0)
        acc[...] = plane.astype(jnp.bfloat16)

        rdmas_b = []
        for i, p in enumerate(peers_b):
            rdma = pltpu.make_async_remote_copy(
                src_ref=acc,
                dst_ref=recv_b.at[my_z],
                send_sem=send_sems_b.at[i],
                recv_sem=recv_sems_b.at[my_z],
                device_id=p,
                device_id_type=pl.DeviceIdType.MESH,
            )
            rdma.start()
            rdmas_b.append(rdma)

        recv_b[my_z, :, :] = acc[...]

        for s in range(N_Z):
            @pl.when(s != my_z)
            def _():
                pltpu.make_async_remote_copy(
                    src_ref=acc,
                    dst_ref=recv_b.at[s],
                    send_sem=send_sems_b.at[0],
                    recv_sem=recv_sems_b.at[s],
                    device_id=(my_x, my_y, my_z),
                    device_id_type=pl.DeviceIdType.MESH,
                ).wait_recv()

        out_ref[...] = jnp.sum(recv_b[...].astype(jnp.float32), axis=0)

        for rdma in rdmas_a + rdmas_b:
            rdma.wait_send()

    return pl.pallas_call(
        body,
        out_shape=jax.ShapeDtypeStruct((2, d), jnp.float32),
        in_specs=[
            pl.BlockSpec(memory_space=pl.ANY),
            pl.BlockSpec(memory_space=pl.ANY),
        ],
        out_specs=pl.BlockSpec(memory_space=pltpu.VMEM),
        scratch_shapes=[
            pltpu.VMEM((rows, d), jnp.float32),
            pltpu.VMEM((rows, d), jnp.float32),
            pltpu.VMEM((2, d), jnp.bfloat16),
            pltpu.VMEM((N_XY, 2, d), jnp.bfloat16),
            pltpu.VMEM((N_Z, 2, d), jnp.bfloat16),
            pltpu.SemaphoreType.DMA((4,)),
            pltpu.SemaphoreType.DMA((len(OFFSETS_XY),)),
            pltpu.SemaphoreType.DMA((N_XY,)),
            pltpu.SemaphoreType.DMA((len(OFFSETS_Z),)),
            pltpu.SemaphoreType.DMA((N_Z,)),
        ],
        compiler_params=pltpu.CompilerParams(collective_id=0),
    )(x, dy)


# device time: 14683 ns/iter; 1.2988x vs baseline; 1.0037x over previous
import jax
import jax.numpy as jnp
from jax import lax
from jax.experimental import pallas as pl
from jax.experimental.pallas import tpu as pltpu

N_X = 2
N_Y = 4
N_Z = 4
N_XY = N_X * N_Y
N_REPL = N_Y * N_Z

OFFSETS_XY = [
    (dx, dy_) for dx in range(N_X) for dy_ in range(N_Y) if (dx, dy_) != (0, 0)
]
OFFSETS_Z = [dz for dz in range(1, N_Z)]


def kernel(x, dy, gamma):
    del gamma
    m, d = x.shape
    rows = m // N_REPL

    def body(x_hbm, dy_hbm, out_ref, xb, dyb, acc, acc32, recv_a, recv_b,
             in_sems, send_sems_a, recv_sems_a, send_sems_b, recv_sems_b):
        my_x = lax.axis_index("x")
        my_y = lax.axis_index("y")
        my_z = lax.axis_index("z")
        gid = my_x * N_Y + my_y

        peers_a = [
            ((my_x + dx) % N_X, (my_y + dy_) % N_Y, my_z)
            for dx, dy_ in OFFSETS_XY
        ]
        peers_b = [(my_x, my_y, (my_z + dz) % N_Z) for dz in OFFSETS_Z]

        barrier = pltpu.get_barrier_semaphore()
        for p in peers_a + peers_b:
            pl.semaphore_signal(
                barrier, inc=1, device_id=p,
                device_id_type=pl.DeviceIdType.MESH,
            )

        r = my_y * N_Z + my_z
        half = rows // 2
        cps = []
        for c in range(2):
            cp_x = pltpu.make_async_copy(
                x_hbm.at[pl.ds(r * rows + c * half, half), :],
                xb.at[pl.ds(c * half, half), :], in_sems.at[2 * c])
            cp_dy = pltpu.make_async_copy(
                dy_hbm.at[pl.ds(r * rows + c * half, half), :],
                dyb.at[pl.ds(c * half, half), :], in_sems.at[2 * c + 1])
            cp_x.start()
            cp_dy.start()
            cps.append((cp_x, cp_dy))

        partial = None
        for c in range(2):
            cps[c][0].wait()
            cps[c][1].wait()
            xv = xb[pl.ds(c * half, half), :]
            dyv = dyb[pl.ds(c * half, half), :]
            mu = jnp.mean(xv, axis=1, keepdims=True)
            var = jnp.mean(xv * xv, axis=1, keepdims=True) - mu * mu
            rstd = lax.rsqrt(var + 1e-5)
            xhat = (xv - mu) * rstd
            dg = jnp.sum(dyv * xhat, axis=0, keepdims=True)
            db = jnp.sum(dyv, axis=0, keepdims=True)
            chunk = jnp.concatenate([dg, db], axis=0)
            partial = chunk if partial is None else partial + chunk
        partial = partial.astype(jnp.bfloat16)
        acc[...] = partial

        pl.semaphore_wait(barrier, len(peers_a) + len(peers_b))

        rdmas_a = []
        for i, p in enumerate(peers_a):
            rdma = pltpu.make_async_remote_copy(
                src_ref=acc,
                dst_ref=recv_a.at[gid],
                send_sem=send_sems_a.at[i],
                recv_sem=recv_sems_a.at[gid],
                device_id=p,
                device_id_type=pl.DeviceIdType.MESH,
            )
            rdma.start()
            rdmas_a.append(rdma)

        acc32[...] = partial.astype(jnp.float32)

        for s in range(N_XY):
            @pl.when(s != gid)
            def _():
                pltpu.make_async_remote_copy(
                    src_ref=acc,
                    dst_ref=recv_a.at[s],
                    send_sem=send_sems_a.at[0],
                    recv_sem=recv_sems_a.at[s],
                    device_id=(my_x, my_y, my_z),
                    device_id_type=pl.DeviceIdType.MESH,
                ).wait_recv()
                acc32[...] = acc32[...] + recv_a[s].astype(jnp.float32)

        acc[...] = acc32[...].astype(jnp.bfloat16)

        rdmas_b = []
        for i, p in enumerate(peers_b):
            rdma = pltpu.make_async_remote_copy(
                src_ref=acc,
                dst_ref=recv_b.at[my_z],
                send_sem=send_sems_b.at[i],
                recv_sem=recv_sems_b.at[my_z],
                device_id=p,
                device_id_type=pl.DeviceIdType.MESH,
            )
            rdma.start()
            rdmas_b.append(rdma)

        out_ref[...] = acc32[...]

        for s in range(N_Z):
            @pl.when(s != my_z)
            def _():
                pltpu.make_async_remote_copy(
                    src_ref=acc,
                    dst_ref=recv_b.at[s],
                    send_sem=send_sems_b.at[0],
                    recv_sem=recv_sems_b.at[s],
                    device_id=(my_x, my_y, my_z),
                    device_id_type=pl.DeviceIdType.MESH,
                ).wait_recv()
                out_ref[...] = out_ref[...] + recv_b[s].astype(jnp.float32)

        for rdma in rdmas_a + rdmas_b:
            rdma.wait_send()

    return pl.pallas_call(
        body,
        out_shape=jax.ShapeDtypeStruct((2, d), jnp.float32),
        in_specs=[
            pl.BlockSpec(memory_space=pl.ANY),
            pl.BlockSpec(memory_space=pl.ANY),
        ],
        out_specs=pl.BlockSpec(memory_space=pltpu.VMEM),
        scratch_shapes=[
            pltpu.VMEM((rows, d), jnp.float32),
            pltpu.VMEM((rows, d), jnp.float32),
            pltpu.VMEM((2, d), jnp.bfloat16),
            pltpu.VMEM((2, d), jnp.float32),
            pltpu.VMEM((N_XY, 2, d), jnp.bfloat16),
            pltpu.VMEM((N_Z, 2, d), jnp.bfloat16),
            pltpu.SemaphoreType.DMA((4,)),
            pltpu.SemaphoreType.DMA((len(OFFSETS_XY),)),
            pltpu.SemaphoreType.DMA((N_XY,)),
            pltpu.SemaphoreType.DMA((len(OFFSETS_Z),)),
            pltpu.SemaphoreType.DMA((N_Z,)),
        ],
        compiler_params=pltpu.CompilerParams(collective_id=0),
    )(x, dy)
